# baseline (speedup 1.0000x reference)
import functools

import jax
import jax.numpy as jnp
from jax import lax
from jax.experimental import pallas as pl
from jax.experimental.pallas import tpu as pltpu
from jax.experimental.pallas import tpu_sc as plsc

_BATCH = 4096
_SEQ_LEN = 200
_EMSIZE = 64
_VOCAB = 1000000

_NC = 2
_NS = 16
_NW = _NC * _NS

_B_TOTAL = _BATCH * _SEQ_LEN
_CHUNK = 128
_B_PER_W = _B_TOTAL // _NW
_N_CHUNKS = _B_PER_W // _CHUNK

_mesh = plsc.VectorSubcoreMesh(core_axis_name="c", subcore_axis_name="s")


@functools.partial(
    pl.kernel,
    out_type=jax.ShapeDtypeStruct((_B_TOTAL, _EMSIZE), jnp.float32),
    mesh=_mesh,
    scratch_types=[
        pltpu.VMEM((_N_CHUNKS, _CHUNK), jnp.int32),
        pltpu.VMEM((_CHUNK, _EMSIZE), jnp.float32),
        pltpu.SemaphoreType.DMA,
    ],
    compiler_params=pltpu.CompilerParams(use_tc_tiling_on_sc=False),
)
def _embed_sc(idx_hbm, table_hbm, out_hbm, idx_v, rows_v, sem):
    wid = lax.axis_index("s") * _NC + lax.axis_index("c")
    chunk0 = wid * _N_CHUNKS
    pltpu.sync_copy(idx_hbm.at[pl.ds(chunk0, _N_CHUNKS), :], idx_v)

    @pl.loop(0, _N_CHUNKS)
    def _chunk_loop(j):
        pltpu.async_copy(table_hbm.at[idx_v.at[j]], rows_v, sem).wait()
        row0 = (chunk0 + j) * _CHUNK
        pltpu.sync_copy(rows_v, out_hbm.at[pl.ds(row0, _CHUNK), :])


def kernel(sequence, src_word_table):
    idx = sequence.reshape(_B_TOTAL // _CHUNK, _CHUNK)
    out = _embed_sc(idx, src_word_table)
    return out.reshape(_BATCH, _SEQ_LEN, _EMSIZE)


# speedup vs baseline: 1.1158x; 1.1158x over previous
import functools

import jax
import jax.numpy as jnp
from jax import lax
from jax.experimental import pallas as pl
from jax.experimental.pallas import tpu as pltpu
from jax.experimental.pallas import tpu_sc as plsc

_BATCH = 4096
_SEQ_LEN = 200
_EMSIZE = 64

_NC = 2
_NS = 16
_NW = _NC * _NS

_B_TOTAL = _BATCH * _SEQ_LEN
_CHUNK = 128
_B_PER_W = _B_TOTAL // _NW
_N_CHUNKS = _B_PER_W // _CHUNK
_NBUF = 8

_mesh = plsc.VectorSubcoreMesh(core_axis_name="c", subcore_axis_name="s")


@functools.partial(
    pl.kernel,
    out_type=jax.ShapeDtypeStruct((_B_TOTAL, _EMSIZE), jnp.float32),
    mesh=_mesh,
    scratch_types=[
        pltpu.VMEM((_N_CHUNKS, _CHUNK), jnp.int32),
        pltpu.VMEM((_NBUF, _CHUNK, _EMSIZE), jnp.float32),
        pltpu.SemaphoreType.DMA((_NBUF,)),
        pltpu.SemaphoreType.DMA((_NBUF,)),
    ],
    compiler_params=pltpu.CompilerParams(use_tc_tiling_on_sc=False),
)
def _embed_sc(idx_hbm, table_hbm, out_hbm, idx_v, rows_v, gsem, wsem):
    wid = lax.axis_index("s") * _NC + lax.axis_index("c")
    chunk0 = wid * _N_CHUNKS
    pltpu.sync_copy(idx_hbm.at[pl.ds(chunk0, _N_CHUNKS), :], idx_v)

    def _gather(j, b):
        pltpu.async_copy(table_hbm.at[idx_v.at[j]], rows_v.at[b], gsem.at[b])

    def _gather_wait(j, b):
        pltpu.make_async_copy(
            table_hbm.at[idx_v.at[j]], rows_v.at[b], gsem.at[b]
        ).wait()

    def _write(j, b):
        row0 = (chunk0 + j) * _CHUNK
        return pltpu.async_copy(
            rows_v.at[b], out_hbm.at[pl.ds(row0, _CHUNK), :], wsem.at[b]
        )

    for b in range(_NBUF):
        _gather(b, b)

    @pl.loop(0, (_N_CHUNKS - _NBUF) // _NBUF)
    def _steady(t):
        j0 = t * _NBUF
        for b in range(_NBUF):
            _gather_wait(j0 + b, b)
            cp = _write(j0 + b, b)
            cp.wait()
            _gather(j0 + b + _NBUF, b)

    j0 = _N_CHUNKS - _NBUF
    for b in range(_NBUF):
        _gather_wait(j0 + b, b)
        _write(j0 + b, b).wait()


def kernel(sequence, src_word_table):
    idx = sequence.reshape(_B_TOTAL // _CHUNK, _CHUNK)
    out = _embed_sc(idx, src_word_table)
    return out.reshape(_BATCH, _SEQ_LEN, _EMSIZE)
